# baseline (speedup 1.0000x reference)
import functools

import jax
import jax.numpy as jnp
from jax import lax
from jax.experimental import pallas as pl
from jax.experimental.pallas import tpu as pltpu
from jax.experimental.pallas import tpu_sc as plsc

N_NODES = 10000
D = 128
HD = D // 2
NC, NS = 2, 16
EW = 20480
C = 128
NCH = EW // C
NCH2 = NCH // 2
ACC = 10240
RPT = ACC // NS
CW = 16
BLK = 1000

_MESH = plsc.VectorSubcoreMesh(core_axis_name="c", subcore_axis_name="s")
_SC_PARAMS = pltpu.CompilerParams(use_tc_tiling_on_sc=False)


def _sc_aggregate(xh_hbm, eidx_hbm, sums_hbm, cnts_hbm,
                  src_v, dst_v, dum_v, buf0_v, buf1_v, ones_v, z16_v,
                  gsem0, gsem1, ssem, csem, acc_sh, cnt_sh):
    cid = lax.axis_index("c")
    sid = lax.axis_index("s")

    pltpu.sync_copy(eidx_hbm.at[sid], src_v)
    pltpu.sync_copy(eidx_hbm.at[NS + sid], dst_v)
    srow = cid * N_NODES

    def _ofs(i, _):
        r = i // (C // 16)
        c = (i % (C // 16)) * 16
        src_v[r, pl.ds(c, 16)] = src_v[r, pl.ds(c, 16)] + srow
        return 0
    lax.fori_loop(0, NCH * (C // 16), _ofs, 0)

    def _dum(i, _):
        dum_v[pl.ds(i * 16, 16)] = jnp.full((16,), N_NODES, jnp.int32)
        return 0
    lax.fori_loop(0, C // 16, _dum, 0)

    def _zrow(i, _):
        r = i // (HD // 16)
        c = (i % (HD // 16)) * 16
        buf0_v[r, pl.ds(c, 16)] = jnp.zeros((16,), jnp.float32)
        buf1_v[r, pl.ds(c, 16)] = jnp.zeros((16,), jnp.float32)
        return 0
    lax.fori_loop(0, C * (HD // 16), _zrow, 0)

    def _orow(i, _):
        ones_v[i, :] = jnp.ones((CW,), jnp.float32)
        return 0
    lax.fori_loop(0, C, _orow, 0)

    def _z16(i, _):
        z16_v[i, :] = jnp.zeros((CW,), jnp.float32)
        return 0
    lax.fori_loop(0, RPT, _z16, 0)

    base = sid * RPT
    for i in range(RPT // C):
        pltpu.sync_copy(buf0_v, acc_sh.at[pl.ds(base + i * C, C)])
    pltpu.sync_copy(z16_v, cnt_sh.at[pl.ds(base, RPT)])
    plsc.subcore_barrier()

    cbase = cid * NCH2
    pltpu.async_copy(xh_hbm.at[src_v.at[0]], buf0_v, gsem0)
    pltpu.async_copy(buf1_v, acc_sh.at[dum_v], ssem, add=True)
    pltpu.async_copy(z16_v.at[pl.ds(0, C)], cnt_sh.at[dum_v], csem, add=True)

    def _pair(jj, _):
        c0 = jj * 2
        pltpu.make_async_copy(xh_hbm.at[src_v.at[c0]], buf0_v, gsem0).wait()
        pltpu.make_async_copy(buf1_v, acc_sh.at[dum_v], ssem).wait()
        pltpu.async_copy(xh_hbm.at[src_v.at[c0 + 1]], buf1_v, gsem1)
        pltpu.async_copy(buf0_v, acc_sh.at[dst_v.at[c0]], ssem, add=True)
        pltpu.make_async_copy(z16_v.at[pl.ds(0, C)], cnt_sh.at[dum_v],
                              csem).wait()
        pltpu.async_copy(ones_v, cnt_sh.at[dst_v.at[cbase + jj]], csem,
                         add=True)
        pltpu.make_async_copy(xh_hbm.at[src_v.at[c0 + 1]], buf1_v, gsem1).wait()
        pltpu.make_async_copy(buf0_v, acc_sh.at[dum_v], ssem).wait()
        c2 = jnp.minimum(c0 + 2, NCH - 1)
        pltpu.async_copy(xh_hbm.at[src_v.at[c2]], buf0_v, gsem0)
        pltpu.async_copy(buf1_v, acc_sh.at[dst_v.at[c0 + 1]], ssem, add=True)
        return 0
    lax.fori_loop(0, NCH // 2, _pair, 0)

    pltpu.make_async_copy(xh_hbm.at[src_v.at[0]], buf0_v, gsem0).wait()
    pltpu.make_async_copy(buf1_v, acc_sh.at[dum_v], ssem).wait()
    pltpu.make_async_copy(z16_v.at[pl.ds(0, C)], cnt_sh.at[dum_v], csem).wait()

    plsc.subcore_barrier()

    pltpu.sync_copy(acc_sh.at[pl.ds(base, RPT)],
                    sums_hbm.at[pl.ds(cid * ACC + base, RPT)])
    pltpu.sync_copy(cnt_sh.at[pl.ds(base, RPT)],
                    cnts_hbm.at[pl.ds(cid * ACC + base, RPT)])


_sc_call = functools.partial(
    pl.kernel,
    mesh=_MESH,
    compiler_params=_SC_PARAMS,
    out_type=[
        jax.ShapeDtypeStruct((NC * ACC, HD), jnp.float32),
        jax.ShapeDtypeStruct((NC * ACC, CW), jnp.float32),
    ],
    scratch_types=[
        pltpu.VMEM((NCH, C), jnp.int32),
        pltpu.VMEM((NCH, C), jnp.int32),
        pltpu.VMEM((C,), jnp.int32),
        pltpu.VMEM((C, HD), jnp.float32),
        pltpu.VMEM((C, HD), jnp.float32),
        pltpu.VMEM((C, CW), jnp.float32),
        pltpu.VMEM((RPT, CW), jnp.float32),
        pltpu.SemaphoreType.DMA,
        pltpu.SemaphoreType.DMA,
        pltpu.SemaphoreType.DMA,
        pltpu.SemaphoreType.DMA,
        pltpu.VMEM_SHARED((ACC, HD), jnp.float32),
        pltpu.VMEM_SHARED((ACC, CW), jnp.float32),
    ],
)(_sc_aggregate)


def _tc_combine(x_ref, p0_ref, p1_ref, c0_ref, c1_ref, ws_ref, wn_ref, o_ref):
    s = jnp.concatenate([p0_ref[0], p1_ref[0]], axis=1)
    cnt = c0_ref[0, :, 0] + c1_ref[0, :, 0]
    mean = s / jnp.maximum(cnt, 1.0)[:, None]
    a = jnp.dot(x_ref[...], ws_ref[...], preferred_element_type=jnp.float32)
    b = jnp.dot(mean, wn_ref[...], preferred_element_type=jnp.float32)
    o_ref[...] = jnp.maximum(jnp.concatenate([a, b], axis=1), 0.0)


def kernel(x, edge_index, W_self, W_neigh):
    ei = edge_index.astype(jnp.int32)
    e = ei.shape[1]
    pad = NS * EW - e
    ei = jnp.concatenate(
        [ei, jnp.stack([jnp.zeros((pad,), jnp.int32),
                        jnp.full((pad,), N_NODES, jnp.int32)])], axis=1)
    eidx = ei.reshape(2 * NS, NCH, C)

    xh = x.reshape(N_NODES, NC, HD).swapaxes(0, 1).reshape(NC * N_NODES, HD)

    sums, cnts = _sc_call(xh, eidx)
    sums = sums.reshape(NC, ACC, HD)
    cnts = cnts.reshape(NC, ACC, CW)

    return pl.pallas_call(
        _tc_combine,
        grid=(N_NODES // BLK,),
        in_specs=[
            pl.BlockSpec((BLK, D), lambda i: (i, 0)),
            pl.BlockSpec((1, BLK, HD), lambda i: (0, i, 0)),
            pl.BlockSpec((1, BLK, HD), lambda i: (1, i, 0)),
            pl.BlockSpec((1, BLK, CW), lambda i: (0, i, 0)),
            pl.BlockSpec((1, BLK, CW), lambda i: (1, i, 0)),
            pl.BlockSpec((D, D), lambda i: (0, 0)),
            pl.BlockSpec((D, D), lambda i: (0, 0)),
        ],
        out_specs=pl.BlockSpec((BLK, 2 * D), lambda i: (i, 0)),
        out_shape=jax.ShapeDtypeStruct((N_NODES, 2 * D), jnp.float32),
    )(x, sums, sums, cnts, cnts, W_self, W_neigh)


# speedup vs baseline: 1.1631x; 1.1631x over previous
import functools

import jax
import jax.numpy as jnp
from jax import lax
from jax.experimental import pallas as pl
from jax.experimental.pallas import tpu as pltpu
from jax.experimental.pallas import tpu_sc as plsc

N_NODES = 10000
D = 128
HD = D // 2
NC, NS = 2, 16
EW = 20480
C = 128
NCH = EW // C
NCH2 = NCH // 2
ACC = 10240
RPT = ACC // NS
CW = 16
BLK = 1000

_MESH = plsc.VectorSubcoreMesh(core_axis_name="c", subcore_axis_name="s")
_SC_PARAMS = pltpu.CompilerParams(use_tc_tiling_on_sc=False,
                                  needs_layout_passes=False)

_PERM = [64 * h + 32 * g + (2 * p if p < 16 else 2 * (p - 16) + 1)
         for h in range(2) for g in range(2) for p in range(32)]


def _sc_aggregate(xh_hbm, eidx_hbm, sums_hbm, cnts_hbm,
                  src_v, dst_v, dum_v, bf0_v, bf1_v, f0_v, f1_v,
                  ones_v, z16_v,
                  gsem0, gsem1, ssem0, ssem1, csem, acc_sh, cnt_sh):
    cid = lax.axis_index("c")
    sid = lax.axis_index("s")

    pltpu.sync_copy(eidx_hbm.at[sid], src_v)
    pltpu.sync_copy(eidx_hbm.at[NS + sid], dst_v)
    srow = cid * N_NODES

    def _ofs(i, _):
        r = i // (C // 16)
        c = (i % (C // 16)) * 16
        src_v[r, pl.ds(c, 16)] = src_v[r, pl.ds(c, 16)] + srow
        return 0
    lax.fori_loop(0, NCH * (C // 16), _ofs, 0)

    def _dum(i, _):
        dum_v[pl.ds(i * 16, 16)] = jnp.full((16,), N_NODES, jnp.int32)
        return 0
    lax.fori_loop(0, C // 16, _dum, 0)

    def _zrow(i, _):
        r = i // (HD // 16)
        c = (i % (HD // 16)) * 16
        f0_v[r, pl.ds(c, 16)] = jnp.zeros((16,), jnp.float32)
        f1_v[r, pl.ds(c, 16)] = jnp.zeros((16,), jnp.float32)
        return 0
    lax.fori_loop(0, C * (HD // 16), _zrow, 0)

    def _orow(i, _):
        ones_v[i, :] = jnp.ones((CW,), jnp.float32)
        return 0
    lax.fori_loop(0, C, _orow, 0)

    def _z16(i, _):
        z16_v[i, :] = jnp.zeros((CW,), jnp.float32)
        return 0
    lax.fori_loop(0, RPT, _z16, 0)

    base = sid * RPT
    for i in range(RPT // C):
        pltpu.sync_copy(f0_v, acc_sh.at[pl.ds(base + i * C, C)])
    pltpu.sync_copy(z16_v, cnt_sh.at[pl.ds(base, RPT)])
    plsc.subcore_barrier()

    def _convert(bf_v, f_v):
        def _row(i, _):
            r = i * 2
            for rr in (r, r + 1):
                for g in (0, 1):
                    a, b = plsc.unpack(bf_v[rr, pl.ds(32 * g, 32)],
                                       format=plsc.PackFormat.INTERLEAVED)
                    f_v[rr, pl.ds(32 * g, 16)] = a
                    f_v[rr, pl.ds(32 * g + 16, 16)] = b
            return 0
        lax.fori_loop(0, C // 2, _row, 0)

    cbase = cid * NCH2
    pltpu.async_copy(xh_hbm.at[src_v.at[0]], bf0_v, gsem0)
    pltpu.async_copy(xh_hbm.at[src_v.at[1]], bf1_v, gsem1)
    pltpu.async_copy(f0_v, acc_sh.at[dum_v], ssem0, add=True)
    pltpu.async_copy(f1_v, acc_sh.at[dum_v], ssem1, add=True)
    pltpu.async_copy(z16_v.at[pl.ds(0, C)], cnt_sh.at[dum_v], csem, add=True)

    def _pair(jj, _):
        c0 = jj * 2
        pltpu.make_async_copy(xh_hbm.at[src_v.at[c0]], bf0_v, gsem0).wait()
        pltpu.make_async_copy(f0_v, acc_sh.at[dum_v], ssem0).wait()
        _convert(bf0_v, f0_v)
        c2 = jnp.minimum(c0 + 2, NCH - 1)
        pltpu.async_copy(xh_hbm.at[src_v.at[c2]], bf0_v, gsem0)
        pltpu.async_copy(f0_v, acc_sh.at[dst_v.at[c0]], ssem0, add=True)
        pltpu.make_async_copy(z16_v.at[pl.ds(0, C)], cnt_sh.at[dum_v],
                              csem).wait()
        pltpu.async_copy(ones_v, cnt_sh.at[dst_v.at[cbase + jj]], csem,
                         add=True)
        pltpu.make_async_copy(xh_hbm.at[src_v.at[c0 + 1]], bf1_v, gsem1).wait()
        pltpu.make_async_copy(f1_v, acc_sh.at[dum_v], ssem1).wait()
        _convert(bf1_v, f1_v)
        c3 = jnp.minimum(c0 + 3, NCH - 1)
        pltpu.async_copy(xh_hbm.at[src_v.at[c3]], bf1_v, gsem1)
        pltpu.async_copy(f1_v, acc_sh.at[dst_v.at[c0 + 1]], ssem1, add=True)
        return 0
    lax.fori_loop(0, NCH // 2, _pair, 0)

    pltpu.make_async_copy(xh_hbm.at[src_v.at[0]], bf0_v, gsem0).wait()
    pltpu.make_async_copy(xh_hbm.at[src_v.at[0]], bf1_v, gsem1).wait()
    pltpu.make_async_copy(f0_v, acc_sh.at[dum_v], ssem0).wait()
    pltpu.make_async_copy(f1_v, acc_sh.at[dum_v], ssem1).wait()
    pltpu.make_async_copy(z16_v.at[pl.ds(0, C)], cnt_sh.at[dum_v], csem).wait()

    plsc.subcore_barrier()

    pltpu.sync_copy(acc_sh.at[pl.ds(base, RPT)],
                    sums_hbm.at[pl.ds(cid * ACC + base, RPT)])
    pltpu.sync_copy(cnt_sh.at[pl.ds(base, RPT)],
                    cnts_hbm.at[pl.ds(cid * ACC + base, RPT)])


_sc_call = functools.partial(
    pl.kernel,
    mesh=_MESH,
    compiler_params=_SC_PARAMS,
    out_type=[
        jax.ShapeDtypeStruct((NC * ACC, HD), jnp.float32),
        jax.ShapeDtypeStruct((NC * ACC, CW), jnp.float32),
    ],
    scratch_types=[
        pltpu.VMEM((NCH, C), jnp.int32),
        pltpu.VMEM((NCH, C), jnp.int32),
        pltpu.VMEM((C,), jnp.int32),
        pltpu.VMEM((C, HD), jnp.bfloat16),
        pltpu.VMEM((C, HD), jnp.bfloat16),
        pltpu.VMEM((C, HD), jnp.float32),
        pltpu.VMEM((C, HD), jnp.float32),
        pltpu.VMEM((C, CW), jnp.float32),
        pltpu.VMEM((RPT, CW), jnp.float32),
        pltpu.SemaphoreType.DMA,
        pltpu.SemaphoreType.DMA,
        pltpu.SemaphoreType.DMA,
        pltpu.SemaphoreType.DMA,
        pltpu.SemaphoreType.DMA,
        pltpu.VMEM_SHARED((ACC, HD), jnp.float32),
        pltpu.VMEM_SHARED((ACC, CW), jnp.float32),
    ],
)(_sc_aggregate)


def _tc_combine(x_ref, p0_ref, p1_ref, c0_ref, c1_ref, ws_ref, wn_ref, o_ref):
    s = jnp.concatenate([p0_ref[0], p1_ref[0]], axis=1)
    cnt = c0_ref[0, :, 0] + c1_ref[0, :, 0]
    mean = s / jnp.maximum(cnt, 1.0)[:, None]
    a = jnp.dot(x_ref[...], ws_ref[...], preferred_element_type=jnp.float32)
    b = jnp.dot(mean, wn_ref[...], preferred_element_type=jnp.float32)
    o_ref[...] = jnp.maximum(jnp.concatenate([a, b], axis=1), 0.0)


def kernel(x, edge_index, W_self, W_neigh):
    ei = edge_index.astype(jnp.int32)
    e = ei.shape[1]
    pad = NS * EW - e
    ei = jnp.concatenate(
        [ei, jnp.stack([jnp.zeros((pad,), jnp.int32),
                        jnp.full((pad,), N_NODES, jnp.int32)])], axis=1)
    eidx = ei.reshape(2 * NS, NCH, C)

    xh = (x.reshape(N_NODES, NC, HD).swapaxes(0, 1)
          .reshape(NC * N_NODES, HD).astype(jnp.bfloat16))

    wn = W_neigh[jnp.array(_PERM), :]

    sums, cnts = _sc_call(xh, eidx)
    sums = sums.reshape(NC, ACC, HD)
    cnts = cnts.reshape(NC, ACC, CW)

    return pl.pallas_call(
        _tc_combine,
        grid=(N_NODES // BLK,),
        in_specs=[
            pl.BlockSpec((BLK, D), lambda i: (i, 0)),
            pl.BlockSpec((1, BLK, HD), lambda i: (0, i, 0)),
            pl.BlockSpec((1, BLK, HD), lambda i: (1, i, 0)),
            pl.BlockSpec((1, BLK, CW), lambda i: (0, i, 0)),
            pl.BlockSpec((1, BLK, CW), lambda i: (1, i, 0)),
            pl.BlockSpec((D, D), lambda i: (0, 0)),
            pl.BlockSpec((D, D), lambda i: (0, 0)),
        ],
        out_specs=pl.BlockSpec((BLK, 2 * D), lambda i: (i, 0)),
        out_shape=jax.ShapeDtypeStruct((N_NODES, 2 * D), jnp.float32),
    )(x, sums, sums, cnts, cnts, W_self, wn)


# speedup vs baseline: 1.1724x; 1.0080x over previous
import functools

import jax
import jax.numpy as jnp
from jax import lax
from jax.experimental import pallas as pl
from jax.experimental.pallas import tpu as pltpu
from jax.experimental.pallas import tpu_sc as plsc

N_NODES = 10000
D = 128
HD = D // 2
NC, NS = 2, 16
EW = 20480
C = 128
NCH = EW // C
NCH2 = NCH // 2
ACC = 10240
RPT = ACC // NS
CW = 16
BLK = 2000

_MESH = plsc.VectorSubcoreMesh(core_axis_name="c", subcore_axis_name="s")
_SC_PARAMS = pltpu.CompilerParams(use_tc_tiling_on_sc=False,
                                  needs_layout_passes=False)

_PERM = [64 * h + 32 * g + (2 * p if p < 16 else 2 * (p - 16) + 1)
         for h in range(2) for g in range(2) for p in range(32)]


def _sc_aggregate(xh_hbm, eidx_hbm, sums_hbm, cnts_hbm,
                  src_v, dst_v, dum_v, bf0_v, bf1_v, f0_v, f1_v,
                  ones_v, z16_v,
                  gsem0, gsem1, ssem0, ssem1, csem, acc_sh, cnt_sh):
    cid = lax.axis_index("c")
    sid = lax.axis_index("s")

    pltpu.sync_copy(eidx_hbm.at[sid], src_v)
    pltpu.sync_copy(eidx_hbm.at[NS + sid], dst_v)
    srow = cid * N_NODES

    def _ofs(i, _):
        r = i // (C // 16)
        c = (i % (C // 16)) * 16
        src_v[r, pl.ds(c, 16)] = src_v[r, pl.ds(c, 16)] + srow
        return 0
    lax.fori_loop(0, NCH * (C // 16), _ofs, 0)

    def _dum(i, _):
        dum_v[pl.ds(i * 16, 16)] = jnp.full((16,), N_NODES, jnp.int32)
        return 0
    lax.fori_loop(0, C // 16, _dum, 0)

    def _zrow(i, _):
        r = i // (HD // 16)
        c = (i % (HD // 16)) * 16
        f0_v[r, pl.ds(c, 16)] = jnp.zeros((16,), jnp.float32)
        f1_v[r, pl.ds(c, 16)] = jnp.zeros((16,), jnp.float32)
        return 0
    lax.fori_loop(0, C * (HD // 16), _zrow, 0)

    def _orow(i, _):
        ones_v[i, :] = jnp.ones((CW,), jnp.float32)
        return 0
    lax.fori_loop(0, C, _orow, 0)

    def _z16(i, _):
        z16_v[i, :] = jnp.zeros((CW,), jnp.float32)
        return 0
    lax.fori_loop(0, RPT, _z16, 0)

    base = sid * RPT
    for i in range(RPT // C):
        pltpu.sync_copy(f0_v, acc_sh.at[pl.ds(base + i * C, C)])
    pltpu.sync_copy(z16_v, cnt_sh.at[pl.ds(base, RPT)])
    plsc.subcore_barrier()

    def _convert(bf_v, f_v):
        def _row(i, _):
            r = i * 2
            for rr in (r, r + 1):
                for g in (0, 1):
                    a, b = plsc.unpack(bf_v[rr, pl.ds(32 * g, 32)],
                                       format=plsc.PackFormat.INTERLEAVED)
                    f_v[rr, pl.ds(32 * g, 16)] = a
                    f_v[rr, pl.ds(32 * g + 16, 16)] = b
            return 0
        lax.fori_loop(0, C // 2, _row, 0)

    cbase = cid * NCH2
    pltpu.async_copy(xh_hbm.at[src_v.at[0]], bf0_v, gsem0)
    pltpu.async_copy(xh_hbm.at[src_v.at[1]], bf1_v, gsem1)
    pltpu.async_copy(f0_v, acc_sh.at[dum_v], ssem0, add=True)
    pltpu.async_copy(f1_v, acc_sh.at[dum_v], ssem1, add=True)
    pltpu.async_copy(z16_v.at[pl.ds(0, C)], cnt_sh.at[dum_v], csem, add=True)

    def _pair(jj, _):
        c0 = jj * 2
        pltpu.make_async_copy(xh_hbm.at[src_v.at[c0]], bf0_v, gsem0).wait()
        pltpu.make_async_copy(f0_v, acc_sh.at[dum_v], ssem0).wait()
        _convert(bf0_v, f0_v)
        c2 = jnp.minimum(c0 + 2, NCH - 1)
        pltpu.async_copy(xh_hbm.at[src_v.at[c2]], bf0_v, gsem0)
        pltpu.async_copy(f0_v, acc_sh.at[dst_v.at[c0]], ssem0, add=True)
        pltpu.make_async_copy(z16_v.at[pl.ds(0, C)], cnt_sh.at[dum_v],
                              csem).wait()
        pltpu.async_copy(ones_v, cnt_sh.at[dst_v.at[cbase + jj]], csem,
                         add=True)
        pltpu.make_async_copy(xh_hbm.at[src_v.at[c0 + 1]], bf1_v, gsem1).wait()
        pltpu.make_async_copy(f1_v, acc_sh.at[dum_v], ssem1).wait()
        _convert(bf1_v, f1_v)
        c3 = jnp.minimum(c0 + 3, NCH - 1)
        pltpu.async_copy(xh_hbm.at[src_v.at[c3]], bf1_v, gsem1)
        pltpu.async_copy(f1_v, acc_sh.at[dst_v.at[c0 + 1]], ssem1, add=True)
        return 0
    lax.fori_loop(0, NCH // 2, _pair, 0)

    pltpu.make_async_copy(xh_hbm.at[src_v.at[0]], bf0_v, gsem0).wait()
    pltpu.make_async_copy(xh_hbm.at[src_v.at[0]], bf1_v, gsem1).wait()
    pltpu.make_async_copy(f0_v, acc_sh.at[dum_v], ssem0).wait()
    pltpu.make_async_copy(f1_v, acc_sh.at[dum_v], ssem1).wait()
    pltpu.make_async_copy(z16_v.at[pl.ds(0, C)], cnt_sh.at[dum_v], csem).wait()

    plsc.subcore_barrier()

    pltpu.sync_copy(acc_sh.at[pl.ds(base, RPT)],
                    sums_hbm.at[pl.ds(cid * ACC + base, RPT)])
    pltpu.sync_copy(cnt_sh.at[pl.ds(base, RPT)],
                    cnts_hbm.at[pl.ds(cid * ACC + base, RPT)])


_sc_call = functools.partial(
    pl.kernel,
    mesh=_MESH,
    compiler_params=_SC_PARAMS,
    out_type=[
        jax.ShapeDtypeStruct((NC * ACC, HD), jnp.float32),
        jax.ShapeDtypeStruct((NC * ACC, CW), jnp.float32),
    ],
    scratch_types=[
        pltpu.VMEM((NCH, C), jnp.int32),
        pltpu.VMEM((NCH, C), jnp.int32),
        pltpu.VMEM((C,), jnp.int32),
        pltpu.VMEM((C, HD), jnp.bfloat16),
        pltpu.VMEM((C, HD), jnp.bfloat16),
        pltpu.VMEM((C, HD), jnp.float32),
        pltpu.VMEM((C, HD), jnp.float32),
        pltpu.VMEM((C, CW), jnp.float32),
        pltpu.VMEM((RPT, CW), jnp.float32),
        pltpu.SemaphoreType.DMA,
        pltpu.SemaphoreType.DMA,
        pltpu.SemaphoreType.DMA,
        pltpu.SemaphoreType.DMA,
        pltpu.SemaphoreType.DMA,
        pltpu.VMEM_SHARED((ACC, HD), jnp.float32),
        pltpu.VMEM_SHARED((ACC, CW), jnp.float32),
    ],
)(_sc_aggregate)


def _tc_combine(x_ref, p0_ref, p1_ref, c0_ref, c1_ref, ws_ref, wn_ref, o_ref):
    s = jnp.concatenate([p0_ref[0], p1_ref[0]], axis=1)
    cnt = c0_ref[0, :, 0] + c1_ref[0, :, 0]
    mean = s / jnp.maximum(cnt, 1.0)[:, None]
    a = jnp.dot(x_ref[...], ws_ref[...], preferred_element_type=jnp.float32)
    b = jnp.dot(mean, wn_ref[...], preferred_element_type=jnp.float32)
    o_ref[...] = jnp.maximum(jnp.concatenate([a, b], axis=1), 0.0)


def kernel(x, edge_index, W_self, W_neigh):
    ei = edge_index.astype(jnp.int32)
    e = ei.shape[1]
    pad = NS * EW - e
    ei = jnp.concatenate(
        [ei, jnp.stack([jnp.zeros((pad,), jnp.int32),
                        jnp.full((pad,), N_NODES, jnp.int32)])], axis=1)
    eidx = ei.reshape(2 * NS, NCH, C)

    xh = (x.reshape(N_NODES, NC, HD).swapaxes(0, 1)
          .reshape(NC * N_NODES, HD).astype(jnp.bfloat16))

    wn = W_neigh[jnp.array(_PERM), :]

    sums, cnts = _sc_call(xh, eidx)
    sums = sums.reshape(NC, ACC, HD)
    cnts = cnts.reshape(NC, ACC, CW)

    return pl.pallas_call(
        _tc_combine,
        grid=(N_NODES // BLK,),
        in_specs=[
            pl.BlockSpec((BLK, D), lambda i: (i, 0)),
            pl.BlockSpec((1, BLK, HD), lambda i: (0, i, 0)),
            pl.BlockSpec((1, BLK, HD), lambda i: (1, i, 0)),
            pl.BlockSpec((1, BLK, CW), lambda i: (0, i, 0)),
            pl.BlockSpec((1, BLK, CW), lambda i: (1, i, 0)),
            pl.BlockSpec((D, D), lambda i: (0, 0)),
            pl.BlockSpec((D, D), lambda i: (0, 0)),
        ],
        out_specs=pl.BlockSpec((BLK, 2 * D), lambda i: (i, 0)),
        out_shape=jax.ShapeDtypeStruct((N_NODES, 2 * D), jnp.float32),
    )(x, sums, sums, cnts, cnts, W_self, wn)
